# baseline (speedup 1.0000x reference)
import functools

import jax
import jax.numpy as jnp
from jax.experimental import pallas as pl
from jax.experimental.pallas import tpu as pltpu

B, T, D_IN, D_OUT, R, E = 32, 1, 4096, 4096, 16, 16
TILE_O = 512


def _fused_body(x_ref, ids_ref, w_ref, a_ref, b_ref, out_ref, mid_ref):
    @pl.when(pl.program_id(0) == 0)
    def _():
        mid_all = jax.lax.dot_general(
            x_ref[...], a_ref[...],
            (((1,), (1,)), ((), ())),
            preferred_element_type=jnp.float32,
        )
        col_e = jax.lax.broadcasted_iota(jnp.int32, (B, E * R), 1) // R
        mask = col_e == ids_ref[...]
        mid_ref[...] = jnp.where(mask, mid_all, 0.0)

    h = jax.lax.dot_general(
        x_ref[...], w_ref[...],
        (((1,), (1,)), ((), ())),
        preferred_element_type=jnp.float32,
    )
    acc = h
    for e in range(E):
        m_e = mid_ref[:, e * R:(e + 1) * R]
        b_e = b_ref[e]
        acc = acc + jax.lax.dot_general(
            m_e, b_e, (((1,), (1,)), ((), ())),
            preferred_element_type=jnp.float32,
        )
    out_ref[...] = acc


@jax.jit
def kernel(x, adapter_ids, W_base, A_all, B_all):
    x2 = x.reshape(B, D_IN)
    a2 = A_all.reshape(E * R, D_IN)
    ids2 = adapter_ids.reshape(B, 1).astype(jnp.int32)
    grid = (D_OUT // TILE_O,)
    out = pl.pallas_call(
        _fused_body,
        grid=grid,
        in_specs=[
            pl.BlockSpec((B, D_IN), lambda j: (0, 0)),
            pl.BlockSpec((B, 1), lambda j: (0, 0)),
            pl.BlockSpec((TILE_O, D_IN), lambda j: (j, 0)),
            pl.BlockSpec((E * R, D_IN), lambda j: (0, 0)),
            pl.BlockSpec((E, TILE_O, R), lambda j: (0, j, 0)),
        ],
        out_specs=pl.BlockSpec((B, TILE_O), lambda j: (0, j)),
        out_shape=jax.ShapeDtypeStruct((B, D_OUT), jnp.float32),
        scratch_shapes=[pltpu.VMEM((B, E * R), jnp.float32)],
    )(x2, ids2, W_base, a2, B_all)
    return out.reshape(B, T, D_OUT)


# speedup vs baseline: 1.5961x; 1.5961x over previous
import jax
import jax.numpy as jnp
from jax.experimental import pallas as pl
from jax.experimental.pallas import tpu as pltpu

B, T, D_IN, D_OUT, R, E = 32, 1, 4096, 4096, 16, 16
TILE_O = 512
NT = D_OUT // TILE_O
NBUF = 4


def _body(x_ref, ids_ref, a_ref, w_hbm, b_hbm, out_ref,
          w_buf, b_vmem, mid_ref, w_sems, b_sem):
    def w_copy(j, slot):
        return pltpu.make_async_copy(
            w_hbm.at[pl.ds(j * TILE_O, TILE_O), :],
            w_buf.at[slot],
            w_sems.at[slot],
        )

    b_copy = pltpu.make_async_copy(b_hbm, b_vmem, b_sem)
    b_copy.start()
    for s in range(NBUF):
        w_copy(s, s).start()

    mid_all = jax.lax.dot_general(
        x_ref[...], a_ref[...], (((1,), (1,)), ((), ())),
        preferred_element_type=jnp.float32,
    )
    col_e = jax.lax.broadcasted_iota(jnp.int32, (B, E * R), 1) // R
    mid_ref[...] = jnp.where(col_e == ids_ref[...], mid_all, 0.0)

    b_copy.wait()
    out_ref[...] = jax.lax.dot_general(
        mid_ref[...], b_vmem[...], (((1,), (0,)), ((), ())),
        preferred_element_type=jnp.float32,
    )

    for j in range(NT):
        slot = j % NBUF
        w_copy(j, slot).wait()
        h = jax.lax.dot_general(
            x_ref[...], w_buf[slot], (((1,), (1,)), ((), ())),
            preferred_element_type=jnp.float32,
        )
        nxt = j + NBUF
        if nxt < NT:
            w_copy(nxt, slot).start()
        out_ref[:, pl.ds(j * TILE_O, TILE_O)] += h


@jax.jit
def kernel(x, adapter_ids, W_base, A_all, B_all):
    x2 = x.reshape(B, D_IN)
    a2 = A_all.reshape(E * R, D_IN)
    b_r = jnp.swapaxes(B_all, 1, 2).reshape(E * R, D_OUT)
    ids2 = adapter_ids.reshape(B, 1).astype(jnp.int32)
    out = pl.pallas_call(
        _body,
        in_specs=[
            pl.BlockSpec((B, D_IN), lambda: (0, 0)),
            pl.BlockSpec((B, 1), lambda: (0, 0)),
            pl.BlockSpec((E * R, D_IN), lambda: (0, 0)),
            pl.BlockSpec(memory_space=pltpu.MemorySpace.HBM),
            pl.BlockSpec(memory_space=pltpu.MemorySpace.HBM),
        ],
        out_specs=pl.BlockSpec((B, D_OUT), lambda: (0, 0)),
        out_shape=jax.ShapeDtypeStruct((B, D_OUT), jnp.float32),
        scratch_shapes=[
            pltpu.VMEM((NBUF, TILE_O, D_IN), jnp.float32),
            pltpu.VMEM((E * R, D_OUT), jnp.float32),
            pltpu.VMEM((B, E * R), jnp.float32),
            pltpu.SemaphoreType.DMA((NBUF,)),
            pltpu.SemaphoreType.DMA,
        ],
    )(x2, ids2, a2, W_base, b_r)
    return out.reshape(B, T, D_OUT)
